# baseline (speedup 1.0000x reference)
import jax
import jax.numpy as jnp
from jax.experimental import pallas as pl

_NUM_TASKS = 4
_HIDDEN = 768
_EPS = 1e-12


def _tc_body(ids_ref, ww_ref, wp_ref, wt_ref, g_ref, b_ref, out_ref):
    s = ww_ref[...] + wt_ref[...] + wp_ref[...]
    mean = jnp.mean(s, axis=-1, keepdims=True)
    var = jnp.mean(jnp.square(s - mean), axis=-1, keepdims=True)
    comb = (s - mean) * jax.lax.rsqrt(var + _EPS) * g_ref[...] + b_ref[...]

    ids = ids_ref[0]
    onehot_t = (jax.lax.broadcasted_iota(jnp.int32, (_NUM_TASKS, ids.shape[1]), 0)
                == ids).astype(jnp.float32)
    out_ref[...] = jax.lax.dot_general(
        onehot_t, comb, (((0,), (0,)), ((), ())),
        preferred_element_type=jnp.float32)


def kernel(input_ids, W_word, W_pos, W_tok, gamma, beta):
    batch, l = input_ids.shape
    n = batch * l
    block = 8192
    grid = n // block
    ids3 = input_ids.reshape(grid, 1, block).astype(jnp.int32)
    g2 = gamma.reshape(1, _HIDDEN)
    b2 = beta.reshape(1, _HIDDEN)

    out = pl.pallas_call(
        _tc_body,
        grid=(grid,),
        in_specs=[
            pl.BlockSpec((1, 1, block), lambda i: (i, 0, 0)),
            pl.BlockSpec((_NUM_TASKS, _HIDDEN), lambda i: (0, 0)),
            pl.BlockSpec((_NUM_TASKS, _HIDDEN), lambda i: (0, 0)),
            pl.BlockSpec((_NUM_TASKS, _HIDDEN), lambda i: (0, 0)),
            pl.BlockSpec((1, _HIDDEN), lambda i: (0, 0)),
            pl.BlockSpec((1, _HIDDEN), lambda i: (0, 0)),
        ],
        out_specs=pl.BlockSpec((block, _HIDDEN), lambda i: (i, 0)),
        out_shape=jax.ShapeDtypeStruct((n, _HIDDEN), jnp.float32),
    )(ids3, W_word, W_pos, W_tok, g2, b2)
    return out.reshape(batch, l, _HIDDEN)


# speedup vs baseline: 1.0104x; 1.0104x over previous
import jax
import jax.numpy as jnp
from jax.experimental import pallas as pl
from jax.experimental.pallas import tpu as pltpu

_NUM_TASKS = 4
_HIDDEN = 768
_EPS = 1e-12
_BLOCK = 2048
_NBUF = 8


def _tc_body(ids_ref, ww_ref, wp_ref, wt_ref, g_ref, b_ref, out_hbm,
             stage, sems):
    s = ww_ref[...] + wt_ref[...] + wp_ref[...]
    mean = jnp.mean(s, axis=-1, keepdims=True)
    var = jnp.mean(jnp.square(s - mean), axis=-1, keepdims=True)
    comb = (s - mean) * jax.lax.rsqrt(var + _EPS) * g_ref[...] + b_ref[...]

    n_blocks = ids_ref.shape[0]

    def step(i, _):
        ids = ids_ref[i]
        onehot_t = (jax.lax.broadcasted_iota(
            jnp.int32, (_NUM_TASKS, _BLOCK), 0) == ids).astype(jnp.float32)
        k = jax.lax.rem(i, _NBUF)
        @pl.when(i >= _NBUF)
        def _():
            pltpu.make_async_copy(
                stage.at[k], out_hbm.at[pl.ds((i - _NBUF) * _BLOCK, _BLOCK)],
                sems.at[k]).wait()
        stage[k] = jax.lax.dot_general(
            onehot_t, comb, (((0,), (0,)), ((), ())),
            preferred_element_type=jnp.float32)
        pltpu.make_async_copy(
            stage.at[k], out_hbm.at[pl.ds(i * _BLOCK, _BLOCK)],
            sems.at[k]).start()
        return 0

    jax.lax.fori_loop(0, n_blocks, step, 0)

    def drain(i, _):
        k = jax.lax.rem(n_blocks + i, _NBUF)
        pltpu.make_async_copy(
            stage.at[k],
            out_hbm.at[pl.ds((n_blocks - _NBUF + i) * _BLOCK, _BLOCK)],
            sems.at[k]).wait()
        return 0

    jax.lax.fori_loop(0, _NBUF, drain, 0)


def kernel(input_ids, W_word, W_pos, W_tok, gamma, beta):
    batch, l = input_ids.shape
    n = batch * l
    n_blocks = n // _BLOCK
    ids3 = input_ids.reshape(n_blocks, 1, _BLOCK).astype(jnp.int32)
    g2 = gamma.reshape(1, _HIDDEN)
    b2 = beta.reshape(1, _HIDDEN)

    out = pl.pallas_call(
        _tc_body,
        in_specs=[
            pl.BlockSpec(memory_space=pltpu.VMEM),
            pl.BlockSpec(memory_space=pltpu.VMEM),
            pl.BlockSpec(memory_space=pltpu.VMEM),
            pl.BlockSpec(memory_space=pltpu.VMEM),
            pl.BlockSpec(memory_space=pltpu.VMEM),
            pl.BlockSpec(memory_space=pltpu.VMEM),
        ],
        out_specs=pl.BlockSpec(memory_space=pl.ANY),
        out_shape=jax.ShapeDtypeStruct((n, _HIDDEN), jnp.float32),
        scratch_shapes=[
            pltpu.VMEM((_NBUF, _BLOCK, _HIDDEN), jnp.float32),
            pltpu.SemaphoreType.DMA((_NBUF,)),
        ],
    )(ids3, W_word, W_pos, W_tok, g2, b2)
    return out.reshape(batch, l, _HIDDEN)
